# Initial kernel scaffold; baseline (speedup 1.0000x reference)
import jax
import jax.numpy as jnp
from jax.experimental import pallas as pl


def kernel(h, src_index, dst_index, gamma, beta, Wv, bv, W1, b1, W2, b2):
    raise NotImplementedError("write your pallas kernel here")


# speedup vs baseline: 2.0172x; 2.0172x over previous
import functools
import math

import jax
import jax.numpy as jnp
from jax.experimental import pallas as pl
from jax.experimental.pallas import tpu as pltpu

_EPS = 1e-5
_INV_SQRT2 = 0.7071067811865476


def _body(x_ref, src_ref, dst_ref, gamma_ref, beta_ref, wcat_ref, bv_ref,
          b1_ref, w2_ref, b2_ref, out_ref, v_scr, a_scr, c_scr, acc_scr,
          *, n_nodes, n_edges, d, gh):
    gamma = gamma_ref[...]
    beta = beta_ref[...]
    wcat = wcat_ref[...]
    bv = bv_ref[...]
    b1 = b1_ref[...]
    w2 = w2_ref[...]
    b2 = b2_ref[0]

    for n in range(n_nodes):
        xn = x_ref[:, n * d:(n + 1) * d]
        mu = jnp.mean(xn, axis=1, keepdims=True)
        xc = xn - mu
        var = jnp.mean(xc * xc, axis=1, keepdims=True)
        xhat = xc * jax.lax.rsqrt(var + _EPS) * gamma + beta
        y = jnp.dot(xhat, wcat, preferred_element_type=jnp.float32,
                    precision=jax.lax.Precision.HIGHEST)
        v_scr[n] = y[:, :d] + bv
        a_scr[n] = y[:, d:d + gh]
        c_scr[n] = y[:, d + gh:] + b1

    acc_scr[...] = jnp.zeros_like(acc_scr)

    w2b = w2.reshape(1, 1, gh)
    for e in range(n_edges):
        s = src_ref[e]
        t = dst_ref[e]
        hin = a_scr[pl.ds(s, 1)] + c_scr[pl.ds(t, 1)]
        gel = 0.5 * hin * (1.0 + jax.lax.erf(hin * _INV_SQRT2))
        u = jnp.sum(gel * w2b, axis=2, keepdims=True) + b2
        gate = jax.nn.sigmoid(u)
        acc_scr[pl.ds(t, 1)] = acc_scr[pl.ds(t, 1)] + gate * v_scr[pl.ds(s, 1)]

    for n in range(n_nodes):
        out_ref[:, n * d:(n + 1) * d] = (
            x_ref[:, n * d:(n + 1) * d] + acc_scr[n])


def kernel(h, src_index, dst_index, gamma, beta, Wv, bv, W1, b1, W2, b2):
    B, N, d = h.shape
    E = src_index.shape[0]
    gh = W1.shape[0]

    wcat = jnp.concatenate([Wv.T, W1[:, :d].T, W1[:, d:].T], axis=1)
    h2 = h.reshape(B, N * d)

    bt = 256
    grid = (B // bt,)

    body = functools.partial(_body, n_nodes=N, n_edges=E, d=d, gh=gh)
    out = pl.pallas_call(
        body,
        grid=grid,
        in_specs=[
            pl.BlockSpec((bt, N * d), lambda i: (i, 0)),
            pl.BlockSpec(memory_space=pltpu.SMEM),
            pl.BlockSpec(memory_space=pltpu.SMEM),
            pl.BlockSpec(memory_space=pltpu.VMEM),
            pl.BlockSpec(memory_space=pltpu.VMEM),
            pl.BlockSpec(memory_space=pltpu.VMEM),
            pl.BlockSpec(memory_space=pltpu.VMEM),
            pl.BlockSpec(memory_space=pltpu.VMEM),
            pl.BlockSpec(memory_space=pltpu.VMEM),
            pl.BlockSpec(memory_space=pltpu.SMEM),
        ],
        out_specs=pl.BlockSpec((bt, N * d), lambda i: (i, 0)),
        out_shape=jax.ShapeDtypeStruct((B, N * d), jnp.float32),
        scratch_shapes=[
            pltpu.VMEM((N, bt, d), jnp.float32),
            pltpu.VMEM((N, bt, gh), jnp.float32),
            pltpu.VMEM((N, bt, gh), jnp.float32),
            pltpu.VMEM((N, bt, d), jnp.float32),
        ],
    )(
        h2,
        src_index,
        dst_index,
        gamma.reshape(1, d),
        beta.reshape(1, d),
        wcat,
        bv.reshape(1, d),
        b1.reshape(1, gh),
        W2.reshape(1, gh),
        b2,
    )
    return out.reshape(B, N, d)


# speedup vs baseline: 2.4980x; 1.2383x over previous
import functools
import math

import jax
import jax.numpy as jnp
from jax.experimental import pallas as pl
from jax.experimental.pallas import tpu as pltpu

_EPS = 1e-5
_INV_SQRT2 = 0.7071067811865476


def _body(x_ref, src_ref, dst_ref, gamma_ref, beta_ref, wcat_ref, bv_ref,
          b1_ref, w2_ref, b2_ref, out_ref, v_scr, a_scr, c_scr, acc_scr,
          *, n_nodes, n_edges, d, gh):
    gamma = gamma_ref[...]
    beta = beta_ref[...]
    wcat = wcat_ref[...]
    bv = bv_ref[...]
    b1 = b1_ref[...]
    w2 = w2_ref[...]
    b2 = b2_ref[0]

    for n in range(n_nodes):
        xn = x_ref[:, n * d:(n + 1) * d]
        mu = jnp.mean(xn, axis=1, keepdims=True)
        xc = xn - mu
        var = jnp.mean(xc * xc, axis=1, keepdims=True)
        xhat = xc * jax.lax.rsqrt(var + _EPS) * gamma + beta
        y = jnp.dot(xhat, wcat, preferred_element_type=jnp.float32)
        v_scr[n] = y[:, :d] + bv
        a_scr[n] = y[:, d:d + gh]
        c_scr[n] = y[:, d + gh:] + b1

    acc_scr[...] = jnp.zeros_like(acc_scr)

    w2b = w2.reshape(1, 1, gh)
    for e in range(n_edges):
        s = src_ref[e]
        t = dst_ref[e]
        hin = a_scr[pl.ds(s, 1)] + c_scr[pl.ds(t, 1)]
        gel = 0.5 * hin * (1.0 + jax.lax.erf(hin * _INV_SQRT2))
        u = jnp.sum(gel * w2b, axis=2, keepdims=True) + b2
        gate = jax.nn.sigmoid(u)
        acc_scr[pl.ds(t, 1)] = acc_scr[pl.ds(t, 1)] + gate * v_scr[pl.ds(s, 1)]

    for n in range(n_nodes):
        out_ref[:, n * d:(n + 1) * d] = (
            x_ref[:, n * d:(n + 1) * d] + acc_scr[n])


def kernel(h, src_index, dst_index, gamma, beta, Wv, bv, W1, b1, W2, b2):
    B, N, d = h.shape
    E = src_index.shape[0]
    gh = W1.shape[0]

    wcat = jnp.concatenate([Wv.T, W1[:, :d].T, W1[:, d:].T], axis=1)
    h2 = h.reshape(B, N * d)

    bt = 256
    grid = (B // bt,)

    body = functools.partial(_body, n_nodes=N, n_edges=E, d=d, gh=gh)
    out = pl.pallas_call(
        body,
        grid=grid,
        in_specs=[
            pl.BlockSpec((bt, N * d), lambda i: (i, 0)),
            pl.BlockSpec(memory_space=pltpu.SMEM),
            pl.BlockSpec(memory_space=pltpu.SMEM),
            pl.BlockSpec(memory_space=pltpu.VMEM),
            pl.BlockSpec(memory_space=pltpu.VMEM),
            pl.BlockSpec(memory_space=pltpu.VMEM),
            pl.BlockSpec(memory_space=pltpu.VMEM),
            pl.BlockSpec(memory_space=pltpu.VMEM),
            pl.BlockSpec(memory_space=pltpu.VMEM),
            pl.BlockSpec(memory_space=pltpu.SMEM),
        ],
        out_specs=pl.BlockSpec((bt, N * d), lambda i: (i, 0)),
        out_shape=jax.ShapeDtypeStruct((B, N * d), jnp.float32),
        scratch_shapes=[
            pltpu.VMEM((N, bt, d), jnp.float32),
            pltpu.VMEM((N, bt, gh), jnp.float32),
            pltpu.VMEM((N, bt, gh), jnp.float32),
            pltpu.VMEM((N, bt, d), jnp.float32),
        ],
    )(
        h2,
        src_index,
        dst_index,
        gamma.reshape(1, d),
        beta.reshape(1, d),
        wcat,
        bv.reshape(1, d),
        b1.reshape(1, gh),
        W2.reshape(1, gh),
        b2,
    )
    return out.reshape(B, N, d)


# speedup vs baseline: 2.7669x; 1.1077x over previous
import functools

import jax
import jax.numpy as jnp
from jax.experimental import pallas as pl
from jax.experimental.pallas import tpu as pltpu

_EPS = 1e-5
_INV_SQRT2 = 0.7071067811865476


def _body(x_ref, src_ref, dst_ref, gamma_ref, beta_ref, wcat_ref, bv_ref,
          b1_ref, w2_ref, b2_ref, out_ref, v_scr, a_scr, c_scr, acc_scr,
          *, n_nodes, n_edges, d, gh):
    gamma = gamma_ref[...]
    beta = beta_ref[...]
    wcat = wcat_ref[...]
    bv = bv_ref[...]
    b1 = b1_ref[...]
    w2 = w2_ref[...]
    b2 = b2_ref[0]

    x = x_ref[...]
    mu = jnp.mean(x, axis=2, keepdims=True)
    xc = x - mu
    var = jnp.mean(xc * xc, axis=2, keepdims=True)
    xhat = (xc * jax.lax.rsqrt(var + _EPS) * gamma.reshape(1, 1, d)
            + beta.reshape(1, 1, d))
    for n in range(n_nodes):
        y = jnp.dot(xhat[:, n, :], wcat,
                    preferred_element_type=jnp.float32)
        v_scr[n] = y[:, :d] + bv
        a_scr[n] = y[:, d:d + gh]
        c_scr[n] = y[:, d + gh:] + b1

    acc_scr[...] = jnp.zeros_like(acc_scr)

    w2b = w2.reshape(1, 1, gh)
    for e in range(n_edges):
        s = src_ref[e]
        t = dst_ref[e]
        hin = a_scr[pl.ds(s, 1)] + c_scr[pl.ds(t, 1)]
        gel = 0.5 * hin * (1.0 + jax.lax.erf(hin * _INV_SQRT2))
        u = jnp.sum(gel * w2b, axis=2, keepdims=True) + b2
        gate = jax.nn.sigmoid(u)
        acc_scr[pl.ds(t, 1)] = acc_scr[pl.ds(t, 1)] + gate * v_scr[pl.ds(s, 1)]

    for n in range(n_nodes):
        out_ref[:, n, :] = x_ref[:, n, :] + acc_scr[n]


def kernel(h, src_index, dst_index, gamma, beta, Wv, bv, W1, b1, W2, b2):
    B, N, d = h.shape
    E = src_index.shape[0]
    gh = W1.shape[0]

    wcat = jnp.concatenate([Wv.T, W1[:, :d].T, W1[:, d:].T], axis=1)

    bt = 256
    grid = (B // bt,)

    body = functools.partial(_body, n_nodes=N, n_edges=E, d=d, gh=gh)
    out = pl.pallas_call(
        body,
        grid=grid,
        in_specs=[
            pl.BlockSpec((bt, N, d), lambda i: (i, 0, 0)),
            pl.BlockSpec(memory_space=pltpu.SMEM),
            pl.BlockSpec(memory_space=pltpu.SMEM),
            pl.BlockSpec(memory_space=pltpu.VMEM),
            pl.BlockSpec(memory_space=pltpu.VMEM),
            pl.BlockSpec(memory_space=pltpu.VMEM),
            pl.BlockSpec(memory_space=pltpu.VMEM),
            pl.BlockSpec(memory_space=pltpu.VMEM),
            pl.BlockSpec(memory_space=pltpu.VMEM),
            pl.BlockSpec(memory_space=pltpu.SMEM),
        ],
        out_specs=pl.BlockSpec((bt, N, d), lambda i: (i, 0, 0)),
        out_shape=jax.ShapeDtypeStruct((B, N, d), jnp.float32),
        scratch_shapes=[
            pltpu.VMEM((N, bt, d), jnp.float32),
            pltpu.VMEM((N, bt, gh), jnp.float32),
            pltpu.VMEM((N, bt, gh), jnp.float32),
            pltpu.VMEM((N, bt, d), jnp.float32),
        ],
    )(
        h,
        src_index,
        dst_index,
        gamma.reshape(1, d),
        beta.reshape(1, d),
        wcat,
        bv.reshape(1, d),
        b1.reshape(1, gh),
        W2.reshape(1, gh),
        b2,
    )
    return out


# speedup vs baseline: 3.1866x; 1.1517x over previous
import functools

import jax
import jax.numpy as jnp
from jax.experimental import pallas as pl
from jax.experimental.pallas import tpu as pltpu

_EPS = 1e-5
_INV_SQRT2 = 0.7071067811865476


def _body(x_ref, src_ref, dst_ref, gamma_ref, beta_ref, wcat_ref, bv_ref,
          b1_ref, w2_ref, b2_ref, out_ref, v_scr, a_scr, c_scr, acc_scr,
          *, n_pad, n_edges, d, gh, bt):
    gamma = gamma_ref[...]
    beta = beta_ref[...]
    wcat = wcat_ref[...]
    bv = bv_ref[...]
    b1 = b1_ref[...]
    w2 = w2_ref[...]
    b2 = b2_ref[0]

    x = x_ref[...]
    mu = jnp.mean(x, axis=2, keepdims=True)
    xc = x - mu
    var = jnp.mean(xc * xc, axis=2, keepdims=True)
    xhat = (xc * jax.lax.rsqrt(var + _EPS) * gamma.reshape(1, 1, d)
            + beta.reshape(1, 1, d))
    xt = jnp.transpose(xhat, (1, 0, 2))
    y = jnp.dot(xt.reshape(n_pad * bt, d), wcat,
                preferred_element_type=jnp.float32)
    y3 = y.reshape(n_pad, bt, d + 2 * gh)
    v_scr[...] = y3[:, :, :d] + bv.reshape(1, 1, d)
    a_scr[...] = y3[:, :, d:d + gh]
    c_scr[...] = y3[:, :, d + gh:] + b1.reshape(1, 1, gh)

    acc_scr[...] = jnp.zeros_like(acc_scr)

    w2b = w2.reshape(1, 1, gh)
    for e in range(n_edges):
        s = src_ref[e]
        t = dst_ref[e]
        hin = a_scr[pl.ds(s, 1)] + c_scr[pl.ds(t, 1)]
        gel = 0.5 * hin * (1.0 + jax.lax.erf(hin * _INV_SQRT2))
        u = jnp.sum(gel * w2b, axis=2, keepdims=True) + b2
        gate = jax.nn.sigmoid(u)
        acc_scr[pl.ds(t, 1)] = acc_scr[pl.ds(t, 1)] + gate * v_scr[pl.ds(s, 1)]

    out_ref[...] = x + jnp.transpose(acc_scr[...], (1, 0, 2))


def kernel(h, src_index, dst_index, gamma, beta, Wv, bv, W1, b1, W2, b2):
    B, N, d = h.shape
    E = src_index.shape[0]
    gh = W1.shape[0]
    n_pad = (N + 7) // 8 * 8

    wcat = jnp.concatenate([Wv.T, W1[:, :d].T, W1[:, d:].T], axis=1)

    bt = 256
    grid = (B // bt,)

    body = functools.partial(_body, n_pad=n_pad, n_edges=E, d=d, gh=gh, bt=bt)
    out = pl.pallas_call(
        body,
        grid=grid,
        in_specs=[
            pl.BlockSpec((bt, n_pad, d), lambda i: (i, 0, 0)),
            pl.BlockSpec(memory_space=pltpu.SMEM),
            pl.BlockSpec(memory_space=pltpu.SMEM),
            pl.BlockSpec(memory_space=pltpu.VMEM),
            pl.BlockSpec(memory_space=pltpu.VMEM),
            pl.BlockSpec(memory_space=pltpu.VMEM),
            pl.BlockSpec(memory_space=pltpu.VMEM),
            pl.BlockSpec(memory_space=pltpu.VMEM),
            pl.BlockSpec(memory_space=pltpu.VMEM),
            pl.BlockSpec(memory_space=pltpu.SMEM),
        ],
        out_specs=pl.BlockSpec((bt, n_pad, d), lambda i: (i, 0, 0)),
        out_shape=jax.ShapeDtypeStruct((B, N, d), jnp.float32),
        scratch_shapes=[
            pltpu.VMEM((n_pad, bt, d), jnp.float32),
            pltpu.VMEM((n_pad, bt, gh), jnp.float32),
            pltpu.VMEM((n_pad, bt, gh), jnp.float32),
            pltpu.VMEM((n_pad, bt, d), jnp.float32),
        ],
    )(
        h,
        src_index,
        dst_index,
        gamma.reshape(1, d),
        beta.reshape(1, d),
        wcat,
        bv.reshape(1, d),
        b1.reshape(1, gh),
        W2.reshape(1, gh),
        b2,
    )
    return out
